# baseline (speedup 1.0000x reference)
import functools
import math

import jax
import jax.numpy as jnp
from jax import lax
from jax.experimental import pallas as pl
from jax.experimental.pallas import tpu as pltpu
from jax.experimental.pallas import tpu_sc as plsc

D_MODEL = 1024
LANES = 16
NUM_CORES = 2
NUM_SUBCORES = 16
NUM_WORKERS = NUM_CORES * NUM_SUBCORES
CHUNK = 8


def _emb_body(xt_hbm, table_hbm, pe_hbm, out_hbm,
              idx_a, idx_b, pe0, pe1,
              ga0, ga1, ga2, ga3, gb0, gb1, gb2, gb3,
              psem0, psem1, gsems0, gsems1, wsems0, wsems1,
              *, batch, seq):
    scale = math.sqrt(D_MODEL)
    pos_per_w = seq // NUM_WORKERS
    n_groups = pos_per_w // CHUNK
    wid = lax.axis_index("s") * NUM_CORES + lax.axis_index("c")
    g0 = wid * n_groups

    idxs = (idx_a, idx_b)
    pes = (pe0, pe1)
    psems = (psem0, psem1)
    sets = ((ga0, ga1, ga2, ga3), (gb0, gb1, gb2, gb3))
    gsems = (gsems0, gsems1)
    wsems = (wsems0, wsems1)

    def pe_slice(g):
        return pe_hbm.at[pl.ds((g0 + g) * CHUNK, CHUNK)]

    def issue_pe(s, g):
        pltpu.async_copy(pe_slice(g), pes[s], psems[s])

    def wait_pe(s):
        pltpu.make_async_copy(pe_slice(0), pes[s], psems[s]).wait()

    def issue_gather(s, b, idx_ref):
        pltpu.async_copy(table_hbm.at[idx_ref.at[b]], sets[s][b],
                         gsems[s].at[b])

    def wait_gather(s, b, idx_ref):
        pltpu.make_async_copy(table_hbm.at[idx_ref.at[0]], sets[s][b],
                              gsems[s].at[b]).wait()

    def issue_write(s, b, g):
        pltpu.async_copy(sets[s][b],
                         out_hbm.at[pl.ds(b * seq + (g0 + g) * CHUNK, CHUNK)],
                         wsems[s].at[b])

    def wait_write(s, b):
        pltpu.make_async_copy(sets[s][b], out_hbm.at[pl.ds(0, CHUNK)],
                              wsems[s].at[b]).wait()

    def compute_group(s):
        bufs = sets[s]
        pe_v = pes[s]

        def row_body(r, _):
            for j in range(D_MODEL // LANES):
                sl = pl.ds(j * LANES, LANES)
                pv = pe_v[r, sl]
                for b in range(batch):
                    bufs[b][r, sl] = bufs[b][r, sl] * scale + pv
            return 0

        lax.fori_loop(0, CHUNK, row_body, 0)

    def run_group(g, s, first, last):
        t = 1 - s
        idx_cur = idxs[s]
        idx_nxt = idxs[t]

        @pl.when(jnp.logical_not(last))
        def _():
            @pl.when(jnp.logical_not(first))
            def _():
                for b in range(batch):
                    wait_write(t, b)
            for b in range(batch):
                issue_gather(t, b, idx_nxt)
            issue_pe(t, g + 1)

        wait_pe(s)
        for b in range(batch):
            wait_gather(s, b, idx_cur)
        compute_group(s)
        for b in range(batch):
            issue_write(s, b, g)

    pltpu.sync_copy(xt_hbm.at[g0], idx_a)
    issue_pe(0, 0)
    for b in range(batch):
        issue_gather(0, b, idx_a)

    def outer(i, _):
        g = 2 * i
        pltpu.sync_copy(xt_hbm.at[g0 + g + 1], idx_b)
        run_group(g, 0, first=(g == 0), last=jnp.bool_(False))
        is_last = g + 1 == n_groups - 1

        @pl.when(jnp.logical_not(is_last))
        def _():
            pltpu.sync_copy(xt_hbm.at[g0 + g + 2], idx_a)
        run_group(g + 1, 1, first=jnp.bool_(False), last=is_last)
        return 0

    lax.fori_loop(0, n_groups // 2, outer, 0)
    for b in range(batch):
        wait_write(0, b)
        wait_write(1, b)


SEQ_SC = 6144


def kernel(x, table, pe):
    batch, seq_full = x.shape
    seq = SEQ_SC
    x_tc = x[:, seq:]
    xt = x[:, :seq].T.reshape(seq // CHUNK, CHUNK, batch).transpose(0, 2, 1)
    pe2d = pe[0, :seq, :]

    mesh = plsc.VectorSubcoreMesh(core_axis_name="c", subcore_axis_name="s")
    k = pl.kernel(
        functools.partial(_emb_body, batch=batch, seq=seq),
        mesh=mesh,
        out_type=jax.ShapeDtypeStruct((batch * seq, D_MODEL), jnp.float32),
        scratch_types=[
            pltpu.VMEM((batch, CHUNK), jnp.int32),
            pltpu.VMEM((batch, CHUNK), jnp.int32),
            pltpu.VMEM((CHUNK, D_MODEL), jnp.float32),
            pltpu.VMEM((CHUNK, D_MODEL), jnp.float32),
            pltpu.VMEM((CHUNK, D_MODEL), jnp.float32),
            pltpu.VMEM((CHUNK, D_MODEL), jnp.float32),
            pltpu.VMEM((CHUNK, D_MODEL), jnp.float32),
            pltpu.VMEM((CHUNK, D_MODEL), jnp.float32),
            pltpu.VMEM((CHUNK, D_MODEL), jnp.float32),
            pltpu.VMEM((CHUNK, D_MODEL), jnp.float32),
            pltpu.VMEM((CHUNK, D_MODEL), jnp.float32),
            pltpu.VMEM((CHUNK, D_MODEL), jnp.float32),
            pltpu.SemaphoreType.DMA,
            pltpu.SemaphoreType.DMA,
            pltpu.SemaphoreType.DMA((4,)),
            pltpu.SemaphoreType.DMA((4,)),
            pltpu.SemaphoreType.DMA((4,)),
            pltpu.SemaphoreType.DMA((4,)),
        ],
    )
    out = k(xt, table, pe2d)
    out_sc = out.reshape(batch, seq, D_MODEL)
    out_tc = (jnp.take(table, x_tc, axis=0) * math.sqrt(D_MODEL)
              + pe[:, seq:seq_full])
    return jnp.concatenate([out_sc, out_tc], axis=1)


# speedup vs baseline: 2.0461x; 2.0461x over previous
import functools
import math

import jax
import jax.numpy as jnp
from jax import lax
from jax.experimental import pallas as pl
from jax.experimental.pallas import tpu as pltpu
from jax.experimental.pallas import tpu_sc as plsc

D_MODEL = 1024
LANES = 16
NUM_CORES = 2
NUM_SUBCORES = 16
NUM_WORKERS = NUM_CORES * NUM_SUBCORES
CHUNK = 8


def _emb_body(xt_hbm, table_hbm, pe_hbm, out_hbm,
              idx_all, pe0, pe1, ga, gb,
              psem0, psem1, gsem0, gsem1, wsems0, wsems1,
              *, batch, seq):
    scale = math.sqrt(D_MODEL)
    pos_per_w = seq // NUM_WORKERS
    n_groups = pos_per_w // CHUNK
    wid = lax.axis_index("s") * NUM_CORES + lax.axis_index("c")
    g0 = wid * n_groups

    pes = (pe0, pe1)
    psems = (psem0, psem1)
    bufs = (ga, gb)
    gsems = (gsem0, gsem1)
    wsems = (wsems0, wsems1)

    def pe_slice(g):
        return pe_hbm.at[pl.ds((g0 + g) * CHUNK, CHUNK)]

    def issue_pe(s, g):
        pltpu.async_copy(pe_slice(g), pes[s], psems[s])

    def wait_pe(s):
        pltpu.make_async_copy(pe_slice(0), pes[s], psems[s]).wait()

    def issue_gather(s, g):
        pltpu.async_copy(table_hbm.at[idx_all.at[g]], bufs[s], gsems[s])

    def wait_gather(s):
        pltpu.make_async_copy(table_hbm.at[idx_all.at[0]], bufs[s],
                              gsems[s]).wait()

    def issue_write(s, b, g):
        pltpu.async_copy(bufs[s].at[pl.ds(b * CHUNK, CHUNK)],
                         out_hbm.at[pl.ds(b * seq + (g0 + g) * CHUNK, CHUNK)],
                         wsems[s].at[b])

    def wait_write(s, b):
        pltpu.make_async_copy(bufs[s].at[pl.ds(0, CHUNK)],
                              out_hbm.at[pl.ds(0, CHUNK)],
                              wsems[s].at[b]).wait()

    def compute_group(s):
        buf = bufs[s]
        pe_v = pes[s]

        def row_body(r, _):
            for j in range(D_MODEL // LANES):
                sl = pl.ds(j * LANES, LANES)
                pv = pe_v[r, sl]
                for b in range(batch):
                    buf[b * CHUNK + r, sl] = buf[b * CHUNK + r, sl] * scale + pv
            return 0

        lax.fori_loop(0, CHUNK, row_body, 0)

    def run_group(g, s, first, last):
        t = 1 - s

        @pl.when(jnp.logical_not(last))
        def _():
            @pl.when(jnp.logical_not(first))
            def _():
                for b in range(batch):
                    wait_write(t, b)
            issue_gather(t, g + 1)
            issue_pe(t, g + 1)

        wait_pe(s)
        wait_gather(s)
        compute_group(s)
        for b in range(batch):
            issue_write(s, b, g)

    pltpu.sync_copy(xt_hbm.at[pl.ds(g0, n_groups)], idx_all)
    issue_pe(0, 0)
    issue_gather(0, 0)

    def outer(i, _):
        g = 2 * i
        run_group(g, 0, first=(g == 0), last=jnp.bool_(False))
        run_group(g + 1, 1, first=jnp.bool_(False),
                  last=(g + 1 == n_groups - 1))
        return 0

    lax.fori_loop(0, n_groups // 2, outer, 0)
    for b in range(batch):
        wait_write(0, b)
        wait_write(1, b)


def kernel(x, table, pe):
    batch, seq = x.shape
    xt = (x.T.reshape(seq // CHUNK, CHUNK, batch)
          .transpose(0, 2, 1).reshape(seq // CHUNK, batch * CHUNK))
    pe2d = pe[0, :seq, :]

    mesh = plsc.VectorSubcoreMesh(core_axis_name="c", subcore_axis_name="s")
    k = pl.kernel(
        functools.partial(_emb_body, batch=batch, seq=seq),
        mesh=mesh,
        out_type=jax.ShapeDtypeStruct((batch * seq, D_MODEL), jnp.float32),
        scratch_types=[
            pltpu.VMEM((seq // NUM_WORKERS // CHUNK, batch * CHUNK),
                       jnp.int32),
            pltpu.VMEM((CHUNK, D_MODEL), jnp.float32),
            pltpu.VMEM((CHUNK, D_MODEL), jnp.float32),
            pltpu.VMEM((batch * CHUNK, D_MODEL), jnp.float32),
            pltpu.VMEM((batch * CHUNK, D_MODEL), jnp.float32),
            pltpu.SemaphoreType.DMA,
            pltpu.SemaphoreType.DMA,
            pltpu.SemaphoreType.DMA,
            pltpu.SemaphoreType.DMA,
            pltpu.SemaphoreType.DMA((4,)),
            pltpu.SemaphoreType.DMA((4,)),
        ],
    )
    out = k(xt, table, pe2d)
    return out.reshape(batch, seq, D_MODEL)
